# Initial kernel scaffold; baseline (speedup 1.0000x reference)
import functools

import jax
import jax.numpy as jnp
from jax import lax
from jax.experimental import pallas as pl
from jax.experimental.pallas import tpu as pltpu
from jax.experimental.pallas import tpu_sc as plsc

N = 10000
F = 128
H = 2
C = 64
HC = H * C
G = 8
NCLS = 40
E = 320000
E_TOT = E + N

NCORE = 2
NSUB = 16
NW = NCORE * NSUB
CH = 128
NCH = -(-E_TOT // (NW * CH))
PW = NCH * CH
E_PAD = PW * NW
ROWS_PER_TILE = N // NSUB

_mesh = plsc.VectorSubcoreMesh(core_axis_name="c", subcore_axis_name="s")


@functools.partial(
    pl.kernel,
    out_type=(
        jax.ShapeDtypeStruct((NCORE, N, HC), jnp.float32),
        jax.ShapeDtypeStruct((NCORE, N, 16), jnp.float32),
    ),
    mesh=_mesh,
    scratch_types=[
        pltpu.VMEM((N,), jnp.float32),
        pltpu.VMEM((N,), jnp.float32),
        pltpu.VMEM((N,), jnp.float32),
        pltpu.VMEM((N,), jnp.float32),
        pltpu.VMEM((1, CH), jnp.int32),
        pltpu.VMEM((1, CH), jnp.int32),
        pltpu.VMEM((CH, HC), jnp.float32),
        pltpu.VMEM((CH, 16), jnp.float32),
        pltpu.VMEM_SHARED((N, HC), jnp.float32),
        pltpu.VMEM_SHARED((N, 16), jnp.float32),
        pltpu.SemaphoreType.DMA,
    ],
)
def _sc_edge_pass(h_hbm, alpha_hbm, src_hbm, dst_hbm, out_m, out_e,
                  as0, as1, ad0, ad1, srcv, dstv, h_buf, ex_buf,
                  acc_m, acc_e, gsem):
    cid = lax.axis_index("c")
    sid = lax.axis_index("s")
    wid = sid * NCORE + cid

    zero16 = jnp.zeros((16,), jnp.float32)
    iota16 = lax.iota(jnp.int32, 16)
    col0 = jnp.zeros((16,), jnp.int32)
    col1 = jnp.ones((16,), jnp.int32)

    @pl.loop(0, CH)
    def _(r):
        for cc in range(HC // 16):
            h_buf[r, pl.ds(cc * 16, 16)] = zero16
        ex_buf[r, pl.ds(0, 16)] = zero16

    r0 = sid * ROWS_PER_TILE
    for off, nrows in ((0, 128), (128, 128), (256, 128), (384, 128), (512, 113)):
        pltpu.sync_copy(h_buf.at[pl.ds(0, nrows)], acc_m.at[pl.ds(r0 + off, nrows)])
        pltpu.sync_copy(ex_buf.at[pl.ds(0, nrows)], acc_e.at[pl.ds(r0 + off, nrows)])

    pltpu.sync_copy(alpha_hbm.at[0], as0)
    pltpu.sync_copy(alpha_hbm.at[1], as1)
    pltpu.sync_copy(alpha_hbm.at[2], ad0)
    pltpu.sync_copy(alpha_hbm.at[3], ad1)

    plsc.subcore_barrier()

    base_w = wid * PW

    @pl.loop(0, NCH)
    def _(ch):
        eb = base_w + ch * CH
        pltpu.sync_copy(src_hbm.at[pl.ds(eb, CH)], srcv.at[0])
        pltpu.sync_copy(dst_hbm.at[pl.ds(eb, CH)], dstv.at[0])
        gcopy = pltpu.async_copy(h_hbm.at[srcv.at[0]], h_buf, gsem)

        @pl.loop(0, CH // 16)
        def _(i):
            sv = srcv[0, pl.ds(i * 16, 16)]
            dv = dstv[0, pl.ds(i * 16, 16)]
            e0 = plsc.load_gather(as0, [sv]) + plsc.load_gather(ad0, [dv])
            e1 = plsc.load_gather(as1, [sv]) + plsc.load_gather(ad1, [dv])
            e0 = jnp.maximum(e0, 0.2 * e0)
            e1 = jnp.maximum(e1, 0.2 * e1)
            x0 = jnp.exp(e0)
            x1 = jnp.exp(e1)
            live = (eb + i * 16 + iota16) < E_TOT
            x0 = jnp.where(live, x0, 0.0)
            x1 = jnp.where(live, x1, 0.0)
            ridx = i * 16 + iota16
            plsc.store_scatter(ex_buf, [ridx, col0], x0)
            plsc.store_scatter(ex_buf, [ridx, col1], x1)

        gcopy.wait()

        @pl.loop(0, CH)
        def _(r):
            ze = jnp.full((16,), r, jnp.int32)
            z0 = plsc.load_gather(ex_buf, [ze, col0])
            z1 = plsc.load_gather(ex_buf, [ze, col1])
            for cc in range(4):
                h_buf[r, pl.ds(cc * 16, 16)] = h_buf[r, pl.ds(cc * 16, 16)] * z0
            for cc in range(4, 8):
                h_buf[r, pl.ds(cc * 16, 16)] = h_buf[r, pl.ds(cc * 16, 16)] * z1

        pltpu.sync_copy(h_buf, acc_m.at[dstv.at[0]], add=True)
        pltpu.sync_copy(ex_buf, acc_e.at[dstv.at[0]], add=True)

    plsc.subcore_barrier()
    pltpu.sync_copy(acc_m.at[pl.ds(r0, ROWS_PER_TILE)],
                    out_m.at[cid, pl.ds(r0, ROWS_PER_TILE)])
    pltpu.sync_copy(acc_e.at[pl.ds(r0, ROWS_PER_TILE)],
                    out_e.at[cid, pl.ds(r0, ROWS_PER_TILE)])


_HI = lax.Precision.HIGHEST


def _tc0_body(x_ref, w_ref, a_ref, h_ref, at_ref):
    h = jnp.dot(x_ref[...], w_ref[...], preferred_element_type=jnp.float32,
                precision=_HI)
    h_ref[...] = h
    at_ref[...] = lax.dot_general(a_ref[...], h, (((0,), (1,)), ((), ())),
                                  preferred_element_type=jnp.float32,
                                  precision=_HI)


def _normalize(accm_ref, acce_ref, b_ref):
    m = accm_ref[0] + accm_ref[1]
    ex = acce_ref[0] + acce_ref[1]
    cols = lax.broadcasted_iota(jnp.int32, (1, HC), 1)
    den = jnp.where(cols < C, ex[:, 0:1], ex[:, 1:2])
    return m / (den + 1e-16) + b_ref[...]


def _mid_body(accm_ref, acce_ref, b_ref, w_ref, a_ref, h_ref, at_ref):
    g = _normalize(accm_ref, acce_ref, b_ref)
    g = jnp.where(g > 0, g, jnp.exp(g) - 1.0)
    h = jnp.dot(g, w_ref[...], preferred_element_type=jnp.float32,
                precision=_HI)
    h_ref[...] = h
    at_ref[...] = lax.dot_general(a_ref[...], h, (((0,), (1,)), ((), ())),
                                  preferred_element_type=jnp.float32,
                                  precision=_HI)


def _fin_body(accm_ref, acce_ref, b_ref, oh_ref, wl_ref, bl_ref, out_ref):
    h = _normalize(accm_ref, acce_ref, b_ref)
    oh = oh_ref[...]
    pooled_s = lax.dot_general(oh, h, (((0,), (0,)), ((), ())),
                               preferred_element_type=jnp.float32,
                               precision=_HI)
    cnt = jnp.sum(oh, axis=0)[:, None]
    pooled = pooled_s / jnp.maximum(cnt, 1.0)
    logits = jnp.dot(pooled, wl_ref[...], preferred_element_type=jnp.float32,
                     precision=_HI) + bl_ref[...]
    mx = jnp.max(logits, axis=1, keepdims=True)
    ez = jnp.exp(logits - mx)
    out_ref[...] = ez / jnp.sum(ez, axis=1, keepdims=True)


_tc0 = pl.pallas_call(
    _tc0_body,
    out_shape=(
        jax.ShapeDtypeStruct((N, HC), jnp.float32),
        jax.ShapeDtypeStruct((8, N), jnp.float32),
    ),
)

_tc_mid = pl.pallas_call(
    _mid_body,
    out_shape=(
        jax.ShapeDtypeStruct((N, HC), jnp.float32),
        jax.ShapeDtypeStruct((8, N), jnp.float32),
    ),
)

_tc_fin = pl.pallas_call(
    _fin_body,
    out_shape=jax.ShapeDtypeStruct((G, NCLS), jnp.float32),
)


def _make_a8(a_s, a_d):
    a8 = jnp.zeros((HC, 8), jnp.float32)
    a8 = a8.at[0:C, 0].set(a_s[0, 0]).at[C:HC, 1].set(a_s[0, 1])
    a8 = a8.at[0:C, 2].set(a_d[0, 0]).at[C:HC, 3].set(a_d[0, 1])
    return a8


def kernel(x, edge_index, edge_attr, batch, W0, a_s0, a_d0, b0,
           W1, a_s1, a_d1, b1, Wl, a_sl, a_dl, bl, W_lin, b_lin):
    del edge_attr

    loop = jnp.arange(N, dtype=jnp.int32)
    pad = jnp.zeros((E_PAD - E_TOT,), jnp.int32)
    src = jnp.concatenate([edge_index[0], loop, pad])
    dst = jnp.concatenate([edge_index[1], loop, pad])

    onehot = (batch[:, None] == jnp.arange(G, dtype=batch.dtype)[None, :])
    onehot = onehot.astype(jnp.float32)

    h, at = _tc0(x, W0, _make_a8(a_s0, a_d0))
    accm, acce = _sc_edge_pass(h, at, src, dst)
    h, at = _tc_mid(accm, acce, b0.reshape(1, HC), W1, _make_a8(a_s1, a_d1))
    accm, acce = _sc_edge_pass(h, at, src, dst)
    h, at = _tc_mid(accm, acce, b1.reshape(1, HC), Wl, _make_a8(a_sl, a_dl))
    accm, acce = _sc_edge_pass(h, at, src, dst)
    return _tc_fin(accm, acce, bl.reshape(1, HC), onehot, W_lin,
                   b_lin.reshape(1, NCLS))


# speedup vs baseline: 57.1150x; 57.1150x over previous
import dataclasses
import functools

import jax
import jax.numpy as jnp
from jax import lax
from jax.experimental import pallas as pl
from jax.experimental.pallas import tpu as pltpu
from jax.experimental.pallas import tpu_sc as plsc

N = 10000
F = 128
H = 2
C = 64
HC = H * C
G = 8
NCLS = 40
E = 320000
E_TOT = E + N

NCORE = 2
NSUB = 16
NW = NCORE * NSUB
CH = 128
NCH = -(-E_TOT // (NW * CH))
PW = NCH * CH
E_PAD = PW * NW
RPT = 624

_mesh = plsc.VectorSubcoreMesh(core_axis_name="c", subcore_axis_name="s")

_sc_params = pltpu.CompilerParams(
    needs_layout_passes=False,
    use_tc_tiling_on_sc=False,
)


@functools.partial(
    pl.kernel,
    out_type=(
        jax.ShapeDtypeStruct((NCORE, N, HC), jnp.float32),
        jax.ShapeDtypeStruct((NCORE, N, 16), jnp.float32),
    ),
    mesh=_mesh,
    scratch_types=[
        pltpu.VMEM((1, CH), jnp.int32),
        pltpu.VMEM((1, CH), jnp.int32),
        pltpu.VMEM((CH, HC), jnp.float32),
        pltpu.VMEM((CH, 16), jnp.float32),
        pltpu.VMEM((CH, 16), jnp.float32),
        pltpu.VMEM((CH, 16), jnp.float32),
        pltpu.VMEM_SHARED((N, HC), jnp.float32),
        pltpu.VMEM_SHARED((N, 16), jnp.float32),
        pltpu.SemaphoreType.DMA,
        pltpu.SemaphoreType.DMA,
    ],
    compiler_params=_sc_params,
)
def _sc_edge_pass(h_hbm, alpha_hbm, src_hbm, dst_hbm, out_m, out_e,
                  srcv, dstv, h_buf, ex_buf, sa_buf, da_buf,
                  acc_m, acc_e, gsem, asem):
    cid = lax.axis_index("c")
    sid = lax.axis_index("s")
    wid = sid * NCORE + cid

    zero16 = jnp.zeros((16,), jnp.float32)
    iota16 = lax.iota(jnp.int32, 16)
    col0 = jnp.zeros((16,), jnp.int32)
    col1 = jnp.ones((16,), jnp.int32)
    col2 = jnp.full((16,), 2, jnp.int32)
    col3 = jnp.full((16,), 3, jnp.int32)

    @pl.loop(0, CH)
    def _(r):
        for cc in range(HC // 16):
            h_buf[r, pl.ds(cc * 16, 16)] = zero16
        ex_buf[r, pl.ds(0, 16)] = zero16

    r0 = sid * RPT
    for off, nrows in ((0, 128), (128, 128), (256, 128), (384, 128), (512, 112)):
        pltpu.sync_copy(h_buf.at[pl.ds(0, nrows)], acc_m.at[pl.ds(r0 + off, nrows)])
        pltpu.sync_copy(ex_buf.at[pl.ds(0, nrows)], acc_e.at[pl.ds(r0 + off, nrows)])

    @pl.when(sid == NSUB - 1)
    def _():
        pltpu.sync_copy(h_buf.at[pl.ds(0, N - NSUB * RPT)],
                        acc_m.at[pl.ds(NSUB * RPT, N - NSUB * RPT)])
        pltpu.sync_copy(ex_buf.at[pl.ds(0, N - NSUB * RPT)],
                        acc_e.at[pl.ds(NSUB * RPT, N - NSUB * RPT)])

    plsc.subcore_barrier()

    base_w = wid * PW

    @pl.loop(0, NCH)
    def _(ch):
        eb = base_w + ch * CH
        pltpu.sync_copy(src_hbm.at[pl.ds(eb, CH)], srcv.at[0])
        pltpu.sync_copy(dst_hbm.at[pl.ds(eb, CH)], dstv.at[0])
        gcopy = pltpu.async_copy(h_hbm.at[srcv.at[0]], h_buf, gsem)
        acopy1 = pltpu.async_copy(alpha_hbm.at[srcv.at[0]], sa_buf, asem)
        acopy2 = pltpu.async_copy(alpha_hbm.at[dstv.at[0]], da_buf, asem)
        acopy1.wait()
        acopy2.wait()

        @pl.loop(0, CH // 16)
        def _(i):
            ridx = i * 16 + iota16
            e0 = (plsc.load_gather(sa_buf, [ridx, col0])
                  + plsc.load_gather(da_buf, [ridx, col2]))
            e1 = (plsc.load_gather(sa_buf, [ridx, col1])
                  + plsc.load_gather(da_buf, [ridx, col3]))
            e0 = jnp.maximum(e0, 0.2 * e0)
            e1 = jnp.maximum(e1, 0.2 * e1)
            x0 = jnp.exp(e0)
            x1 = jnp.exp(e1)
            live = (eb + ridx) < E_TOT
            x0 = jnp.where(live, x0, 0.0)
            x1 = jnp.where(live, x1, 0.0)
            plsc.store_scatter(ex_buf, [ridx, col0], x0)
            plsc.store_scatter(ex_buf, [ridx, col1], x1)

        gcopy.wait()

        @pl.loop(0, CH)
        def _(r):
            ze = jnp.full((16,), r, jnp.int32)
            z0 = plsc.load_gather(ex_buf, [ze, col0])
            z1 = plsc.load_gather(ex_buf, [ze, col1])
            for cc in range(4):
                h_buf[r, pl.ds(cc * 16, 16)] = h_buf[r, pl.ds(cc * 16, 16)] * z0
            for cc in range(4, 8):
                h_buf[r, pl.ds(cc * 16, 16)] = h_buf[r, pl.ds(cc * 16, 16)] * z1

        pltpu.sync_copy(h_buf, acc_m.at[dstv.at[0]], add=True)
        pltpu.sync_copy(ex_buf, acc_e.at[dstv.at[0]], add=True)

    plsc.subcore_barrier()
    pltpu.sync_copy(acc_m.at[pl.ds(r0, RPT)], out_m.at[cid, pl.ds(r0, RPT)])
    pltpu.sync_copy(acc_e.at[pl.ds(r0, RPT)], out_e.at[cid, pl.ds(r0, RPT)])

    @pl.when(sid == NSUB - 1)
    def _():
        rr = NSUB * RPT
        pltpu.sync_copy(acc_m.at[pl.ds(rr, N - rr)],
                        out_m.at[cid, pl.ds(rr, N - rr)])
        pltpu.sync_copy(acc_e.at[pl.ds(rr, N - rr)],
                        out_e.at[cid, pl.ds(rr, N - rr)])


_HI = lax.Precision.HIGHEST


def _tc0_body(x_ref, w_ref, a_ref, h_ref, at_ref):
    h = jnp.dot(x_ref[...], w_ref[...], preferred_element_type=jnp.float32,
                precision=_HI)
    h_ref[...] = h
    at_ref[...] = lax.dot_general(a_ref[...], h, (((0,), (1,)), ((), ())),
                                  preferred_element_type=jnp.float32,
                                  precision=_HI)


def _normalize(accm_ref, acce_ref, b_ref):
    m = accm_ref[0] + accm_ref[1]
    ex = acce_ref[0] + acce_ref[1]
    cols = lax.broadcasted_iota(jnp.int32, (1, HC), 1)
    den = jnp.where(cols < C, ex[:, 0:1], ex[:, 1:2])
    return m / (den + 1e-16) + b_ref[...]


def _mid_body(accm_ref, acce_ref, b_ref, w_ref, a_ref, h_ref, at_ref):
    g = _normalize(accm_ref, acce_ref, b_ref)
    g = jnp.where(g > 0, g, jnp.exp(g) - 1.0)
    h = jnp.dot(g, w_ref[...], preferred_element_type=jnp.float32,
                precision=_HI)
    h_ref[...] = h
    at_ref[...] = lax.dot_general(a_ref[...], h, (((0,), (1,)), ((), ())),
                                  preferred_element_type=jnp.float32,
                                  precision=_HI)


def _fin_body(accm_ref, acce_ref, b_ref, oh_ref, wl_ref, bl_ref, out_ref):
    h = _normalize(accm_ref, acce_ref, b_ref)
    oh = oh_ref[...]
    pooled_s = lax.dot_general(oh, h, (((0,), (0,)), ((), ())),
                               preferred_element_type=jnp.float32,
                               precision=_HI)
    cnt = jnp.sum(oh, axis=0)[:, None]
    pooled = pooled_s / jnp.maximum(cnt, 1.0)
    logits = jnp.dot(pooled, wl_ref[...], preferred_element_type=jnp.float32,
                     precision=_HI) + bl_ref[...]
    mx = jnp.max(logits, axis=1, keepdims=True)
    ez = jnp.exp(logits - mx)
    out_ref[...] = ez / jnp.sum(ez, axis=1, keepdims=True)


_tc0 = pl.pallas_call(
    _tc0_body,
    out_shape=(
        jax.ShapeDtypeStruct((N, HC), jnp.float32),
        jax.ShapeDtypeStruct((8, N), jnp.float32),
    ),
)

_tc_mid = pl.pallas_call(
    _mid_body,
    out_shape=(
        jax.ShapeDtypeStruct((N, HC), jnp.float32),
        jax.ShapeDtypeStruct((8, N), jnp.float32),
    ),
)

_tc_fin = pl.pallas_call(
    _fin_body,
    out_shape=jax.ShapeDtypeStruct((G, NCLS), jnp.float32),
)


def _make_a8(a_s, a_d):
    a8 = jnp.zeros((HC, 8), jnp.float32)
    a8 = a8.at[0:C, 0].set(a_s[0, 0]).at[C:HC, 1].set(a_s[0, 1])
    a8 = a8.at[0:C, 2].set(a_d[0, 0]).at[C:HC, 3].set(a_d[0, 1])
    return a8


def kernel(x, edge_index, edge_attr, batch, W0, a_s0, a_d0, b0,
           W1, a_s1, a_d1, b1, Wl, a_sl, a_dl, bl, W_lin, b_lin):
    del edge_attr

    loop = jnp.arange(N, dtype=jnp.int32)
    pad = jnp.zeros((E_PAD - E_TOT,), jnp.int32)
    src = jnp.concatenate([edge_index[0], loop, pad])
    dst = jnp.concatenate([edge_index[1], loop, pad])

    onehot = (batch[:, None] == jnp.arange(G, dtype=batch.dtype)[None, :])
    onehot = onehot.astype(jnp.float32)

    def alpha16(at):
        return jnp.pad(at[:4].T, ((0, 0), (0, 12)))

    h, at = _tc0(x, W0, _make_a8(a_s0, a_d0))
    accm, acce = _sc_edge_pass(h, alpha16(at), src, dst)
    h, at = _tc_mid(accm, acce, b0.reshape(1, HC), W1, _make_a8(a_s1, a_d1))
    accm, acce = _sc_edge_pass(h, alpha16(at), src, dst)
    h, at = _tc_mid(accm, acce, b1.reshape(1, HC), Wl, _make_a8(a_sl, a_dl))
    accm, acce = _sc_edge_pass(h, alpha16(at), src, dst)
    return _tc_fin(accm, acce, bl.reshape(1, HC), onehot, W_lin,
                   b_lin.reshape(1, NCLS))
